# baseline (speedup 1.0000x reference)
import functools

import jax
import jax.numpy as jnp
from jax import lax
from jax.experimental import pallas as pl
from jax.experimental.pallas import tpu as pltpu
from jax.experimental.pallas import tpu_sc as plsc

N = 10000
E = 160000
D_H = 512
D_OUT = 128
G = 64
L = 4

NUM_TILES = 16
EDGE_BATCH = 40


@functools.lru_cache(maxsize=None)
def _make_sc_agg(C):
    CPC = C // 2
    EPT = E // NUM_TILES
    NIT = EPT // EDGE_BATCH
    RPT = 624
    TAIL0 = RPT * NUM_TILES
    TAILN = N - TAIL0

    mesh = plsc.VectorSubcoreMesh(core_axis_name="c", subcore_axis_name="s")

    NSET = 5
    NB = 2 * NSET
    NBODY = NIT // NB
    NTAIL = NIT - NBODY * NB
    assert NTAIL in (0, NSET)
    NROW = 9

    @functools.partial(
        pl.kernel,
        out_type=jax.ShapeDtypeStruct((C, N, 128), jnp.float32),
        mesh=mesh,
        scratch_types=(
            [pltpu.VMEM((EDGE_BATCH,), jnp.int32)] * (4 * NSET)
            + [pltpu.VMEM((EDGE_BATCH, 128), jnp.float32)] * NROW
            + [pltpu.VMEM_SHARED((N, 128), jnp.float32)]
            + [pltpu.SemaphoreType.DMA] * (2 + 2 * NROW)
        ),
    )
    def sc_agg(hc_hbm, srcc_hbm, dst_hbm, out_hbm, *scratch):
        sidx = list(scratch[0:NSET]) + list(scratch[2 * NSET:3 * NSET])
        didx = list(scratch[NSET:2 * NSET]) + list(scratch[3 * NSET:4 * NSET])
        rows = scratch[4 * NSET:4 * NSET + NROW]
        acc = scratch[4 * NSET + NROW]
        isem0, isem1 = scratch[4 * NSET + NROW + 1:4 * NSET + NROW + 3]
        ssems = scratch[4 * NSET + NROW + 3:4 * NSET + NROW + 3 + NROW]
        gsems = scratch[4 * NSET + NROW + 3 + NROW:]
        core = lax.axis_index("c")
        sub = lax.axis_index("s")
        r0 = sub * RPT
        e0 = sub * EPT

        for j in range(CPC):
            cc = core * CPC + j

            def load_idx(batch0, half, isem):
                for b in range(NSET):
                    e = e0 + (batch0 + b) * EDGE_BATCH
                    pltpu.async_copy(
                        srcc_hbm.at[pl.ds(cc * E + e, EDGE_BATCH)],
                        sidx[half * NSET + b], isem)
                    pltpu.async_copy(
                        dst_hbm.at[pl.ds(e, EDGE_BATCH)],
                        didx[half * NSET + b], isem)

            def drain_idx(isem):
                for _ in range(2 * NSET):
                    pltpu.make_async_copy(
                        dst_hbm.at[pl.ds(0, EDGE_BATCH)], didx[0], isem
                    ).wait()

            pltpu.sync_copy(
                hc_hbm.at[pl.ds(cc * N + r0, RPT)],
                acc.at[pl.ds(r0, RPT)],
            )

            @pl.when(sub == NUM_TILES - 1)
            def _():
                pltpu.sync_copy(
                    hc_hbm.at[pl.ds(cc * N + TAIL0, TAILN)],
                    acc.at[pl.ds(TAIL0, TAILN)],
                )

            plsc.subcore_barrier()
            load_idx(0, 0, isem0)

            def run_batches(nb, half1_load_base):
                drain_idx(isem0)
                gd, sd = {}, {}
                for i in range(min(NSET, nb)):
                    gd[i] = pltpu.async_copy(
                        hc_hbm.at[sidx[i]], rows[i], gsems[i])
                if nb > NSET:
                    load_idx(half1_load_base, 1, isem1)
                    drain_idx(isem1)
                    for i in range(NSET, min(NROW, nb)):
                        gd[i] = pltpu.async_copy(
                            hc_hbm.at[sidx[i]], rows[i], gsems[i])
                for i in range(nb):
                    gd[i].wait()
                    sd[i] = pltpu.async_copy(
                        rows[i % NROW], acc.at[didx[i]], ssems[i % NROW],
                        add=True)
                    jj = i + NROW
                    if jj < nb:
                        sd[i].wait()
                        gd[jj] = pltpu.async_copy(
                            hc_hbm.at[sidx[jj]], rows[jj % NROW], gsems[jj % NROW])
                for i in range(max(0, nb - NROW), nb):
                    sd[i].wait()

            def body(t, _):
                base = t * NB
                run_batches(NB, base + NSET)

                @pl.when(t < NBODY - 1)
                def _():
                    load_idx(base + NB, 0, isem0)

                return ()

            lax.fori_loop(0, NBODY, body, ())
            if NTAIL:
                load_idx(NBODY * NB, 0, isem0)
                run_batches(NTAIL, 0)
            plsc.subcore_barrier()
            pltpu.sync_copy(
                acc.at[pl.ds(r0, RPT)], out_hbm.at[cc, pl.ds(r0, RPT)]
            )

            @pl.when(sub == NUM_TILES - 1)
            def _():
                pltpu.sync_copy(
                    acc.at[pl.ds(TAIL0, TAILN)],
                    out_hbm.at[cc, pl.ds(TAIL0, TAILN)],
                )

            plsc.subcore_barrier()

    return sc_agg


ROW_BLK = 2000


def _k1_body(z_ref, w_ref, b_ref, out_ref, s1_ref, s2_ref, *, C):
    i = pl.program_id(0)
    acc = jnp.dot(z_ref[0], w_ref[0], preferred_element_type=jnp.float32)
    for c in range(1, C):
        acc += jnp.dot(z_ref[c], w_ref[c], preferred_element_type=jnp.float32)
    z = acc + b_ref[...]
    out_ref[...] = z
    p1 = jnp.sum(z.reshape(ROW_BLK // 8, 8, D_H), axis=0)
    p2 = jnp.sum((z * z).reshape(ROW_BLK // 8, 8, D_H), axis=0)

    @pl.when(i == 0)
    def _():
        s1_ref[...] = p1
        s2_ref[...] = p2

    @pl.when(i > 0)
    def _():
        s1_ref[...] += p1
        s2_ref[...] += p2


def _k1(z_chunks, w1r, b1):
    C = z_chunks.shape[0]
    grid = (N // ROW_BLK,)
    return pl.pallas_call(
        functools.partial(_k1_body, C=C),
        grid=grid,
        in_specs=[
            pl.BlockSpec((C, ROW_BLK, 128), lambda i: (0, i, 0)),
            pl.BlockSpec((C, 128, D_H), lambda i: (0, 0, 0)),
            pl.BlockSpec((1, D_H), lambda i: (0, 0)),
        ],
        out_specs=[
            pl.BlockSpec((ROW_BLK, D_H), lambda i: (i, 0)),
            pl.BlockSpec((8, D_H), lambda i: (0, 0)),
            pl.BlockSpec((8, D_H), lambda i: (0, 0)),
        ],
        out_shape=[
            jax.ShapeDtypeStruct((N, D_H), jnp.float32),
            jax.ShapeDtypeStruct((8, D_H), jnp.float32),
            jax.ShapeDtypeStruct((8, D_H), jnp.float32),
        ],
    )(z_chunks, w1r, b1)


def _k2_body(z_ref, s1_ref, s2_ref, g_ref, be_ref, w2_ref, b2_ref, batch_ref,
             h_ref, pool_ref):
    i = pl.program_id(0)
    s1 = jnp.sum(s1_ref[...], axis=0, keepdims=True)
    s2 = jnp.sum(s2_ref[...], axis=0, keepdims=True)
    mean = s1 * (1.0 / N)
    var = s2 * (1.0 / N) - mean * mean
    inv = lax.rsqrt(var + 1e-5)
    scale = g_ref[...] * inv
    shift = be_ref[...] - mean * scale
    z = z_ref[...]
    r = jnp.maximum(z * scale + shift, 0.0)
    h = jnp.dot(r, w2_ref[...], preferred_element_type=jnp.float32) + b2_ref[...]
    h = jnp.maximum(h, 0.0)
    for c in range(D_H // 128):
        h_ref[c] = h[:, c * 128:(c + 1) * 128]
    onehot = (batch_ref[...] == lax.broadcasted_iota(jnp.int32, (ROW_BLK, G), 1)
              ).astype(jnp.float32)
    pp = lax.dot_general(onehot, h, dimension_numbers=(((0,), (0,)), ((), ())),
                         preferred_element_type=jnp.float32)

    @pl.when(i == 0)
    def _():
        pool_ref[...] = pp

    @pl.when(i > 0)
    def _():
        pool_ref[...] += pp


def _k2(z, s1, s2, g, be, w2, b2, batch2):
    grid = (N // ROW_BLK,)
    return pl.pallas_call(
        _k2_body,
        grid=grid,
        in_specs=[
            pl.BlockSpec((ROW_BLK, D_H), lambda i: (i, 0)),
            pl.BlockSpec((8, D_H), lambda i: (0, 0)),
            pl.BlockSpec((8, D_H), lambda i: (0, 0)),
            pl.BlockSpec((1, D_H), lambda i: (0, 0)),
            pl.BlockSpec((1, D_H), lambda i: (0, 0)),
            pl.BlockSpec((D_H, D_H), lambda i: (0, 0)),
            pl.BlockSpec((1, D_H), lambda i: (0, 0)),
            pl.BlockSpec((ROW_BLK, 1), lambda i: (i, 0)),
        ],
        out_specs=[
            pl.BlockSpec((D_H // 128, ROW_BLK, 128), lambda i: (0, i, 0)),
            pl.BlockSpec((G, D_H), lambda i: (0, 0)),
        ],
        out_shape=[
            jax.ShapeDtypeStruct((D_H // 128, N, 128), jnp.float32),
            jax.ShapeDtypeStruct((G, D_H), jnp.float32),
        ],
    )(z, s1, s2, g, be, w2, b2, batch2)


def _k3_body(p_ref, w_ref, b_ref, out_ref):
    o = jnp.sum(p_ref[...], axis=0)
    logits = jnp.dot(o, w_ref[...], preferred_element_type=jnp.float32) + b_ref[...]
    m = jnp.max(logits, axis=1, keepdims=True)
    e = jnp.exp(logits - m)
    s = jnp.sum(e, axis=1, keepdims=True)
    out_ref[...] = logits - m - jnp.log(s)


def _k3(pools, w_out, b_out):
    return pl.pallas_call(
        _k3_body,
        out_shape=jax.ShapeDtypeStruct((G, D_OUT), jnp.float32),
    )(pools, w_out, b_out)


def kernel(x, params, edge_index, batch):
    src = edge_index[0]
    dst = edge_index[1]
    batch2 = batch.reshape(N, 1)
    hc = x.reshape(N, 2, 128).transpose(1, 0, 2).reshape(2 * N, 128)
    pools = []
    for l in range(L):
        p = params['layers'][l]
        C = hc.shape[0] // N
        srcc = (src[None, :]
                + N * jnp.arange(C, dtype=jnp.int32)[:, None]).reshape(C * E)
        z_chunks = _make_sc_agg(C)(hc, srcc, dst)
        w1r = p['W1'].reshape(C, 128, D_H)
        b1 = p['b1'].reshape(1, D_H)
        zz, s1, s2 = _k1(z_chunks, w1r, b1)
        hcs, pool = _k2(zz, s1, s2, p['g1'].reshape(1, D_H),
                        p['be1'].reshape(1, D_H), p['W2'],
                        p['b2'].reshape(1, D_H), batch2)
        hc = hcs.reshape((D_H // 128) * N, 128)
        pools.append(pool)
    pstack = jnp.stack(pools)
    return _k3(pstack, params['W_out'], params['b_out'].reshape(1, D_OUT))


# speedup vs baseline: 1.0464x; 1.0464x over previous
import functools

import jax
import jax.numpy as jnp
from jax import lax
from jax.experimental import pallas as pl
from jax.experimental.pallas import tpu as pltpu
from jax.experimental.pallas import tpu_sc as plsc

N = 10000
E = 160000
D_H = 512
D_OUT = 128
G = 64
L = 4

NUM_TILES = 16
EDGE_BATCH = 80


@functools.lru_cache(maxsize=None)
def _make_sc_agg(C):
    CPC = C // 2
    EPT = E // NUM_TILES
    NIT = EPT // EDGE_BATCH
    RPT = 624
    TAIL0 = RPT * NUM_TILES
    TAILN = N - TAIL0

    mesh = plsc.VectorSubcoreMesh(core_axis_name="c", subcore_axis_name="s")

    NSET = 5
    NB = 2 * NSET
    NBODY = NIT // NB
    NTAIL = NIT - NBODY * NB
    assert NTAIL in (0, NSET)
    NROW = 4

    @functools.partial(
        pl.kernel,
        out_type=jax.ShapeDtypeStruct((C, N, 128), jnp.float32),
        mesh=mesh,
        scratch_types=(
            [pltpu.VMEM((EDGE_BATCH,), jnp.int32)] * (4 * NSET)
            + [pltpu.VMEM((EDGE_BATCH, 128), jnp.float32)] * NROW
            + [pltpu.VMEM_SHARED((N, 128), jnp.float32)]
            + [pltpu.SemaphoreType.DMA] * (2 + 2 * NROW)
        ),
    )
    def sc_agg(hc_hbm, srcc_hbm, dst_hbm, out_hbm, *scratch):
        sidx = list(scratch[0:NSET]) + list(scratch[2 * NSET:3 * NSET])
        didx = list(scratch[NSET:2 * NSET]) + list(scratch[3 * NSET:4 * NSET])
        rows = scratch[4 * NSET:4 * NSET + NROW]
        acc = scratch[4 * NSET + NROW]
        isem0, isem1 = scratch[4 * NSET + NROW + 1:4 * NSET + NROW + 3]
        ssems = scratch[4 * NSET + NROW + 3:4 * NSET + NROW + 3 + NROW]
        gsems = scratch[4 * NSET + NROW + 3 + NROW:]
        core = lax.axis_index("c")
        sub = lax.axis_index("s")
        r0 = sub * RPT
        e0 = sub * EPT

        for j in range(CPC):
            cc = core * CPC + j

            def load_idx(batch0, half, isem):
                for b in range(NSET):
                    e = e0 + (batch0 + b) * EDGE_BATCH
                    pltpu.async_copy(
                        srcc_hbm.at[pl.ds(cc * E + e, EDGE_BATCH)],
                        sidx[half * NSET + b], isem)
                    pltpu.async_copy(
                        dst_hbm.at[pl.ds(e, EDGE_BATCH)],
                        didx[half * NSET + b], isem)

            def drain_idx(isem):
                for _ in range(2 * NSET):
                    pltpu.make_async_copy(
                        dst_hbm.at[pl.ds(0, EDGE_BATCH)], didx[0], isem
                    ).wait()

            pltpu.sync_copy(
                hc_hbm.at[pl.ds(cc * N + r0, RPT)],
                acc.at[pl.ds(r0, RPT)],
            )

            @pl.when(sub == NUM_TILES - 1)
            def _():
                pltpu.sync_copy(
                    hc_hbm.at[pl.ds(cc * N + TAIL0, TAILN)],
                    acc.at[pl.ds(TAIL0, TAILN)],
                )

            plsc.subcore_barrier()
            load_idx(0, 0, isem0)

            def run_batches(nb, half1_load_base):
                drain_idx(isem0)
                gd, sd = {}, {}
                for i in range(min(NROW, nb)):
                    gd[i] = pltpu.async_copy(
                        hc_hbm.at[sidx[i]], rows[i], gsems[i])
                if nb > NSET:
                    load_idx(half1_load_base, 1, isem1)
                for i in range(nb):
                    gd[i].wait()
                    sd[i] = pltpu.async_copy(
                        rows[i % NROW], acc.at[didx[i]], ssems[i % NROW],
                        add=True)
                    jj = i - 1 + NROW
                    if i >= 1 and jj < nb:
                        if jj == NSET:
                            drain_idx(isem1)
                        sd[i - 1].wait()
                        gd[jj] = pltpu.async_copy(
                            hc_hbm.at[sidx[jj]], rows[jj % NROW], gsems[jj % NROW])
                for i in range(max(0, nb - NROW), nb):
                    sd[i].wait()

            def body(t, _):
                base = t * NB
                run_batches(NB, base + NSET)

                @pl.when(t < NBODY - 1)
                def _():
                    load_idx(base + NB, 0, isem0)

                return ()

            lax.fori_loop(0, NBODY, body, ())
            if NTAIL:
                load_idx(NBODY * NB, 0, isem0)
                run_batches(NTAIL, 0)
            plsc.subcore_barrier()
            pltpu.sync_copy(
                acc.at[pl.ds(r0, RPT)], out_hbm.at[cc, pl.ds(r0, RPT)]
            )

            @pl.when(sub == NUM_TILES - 1)
            def _():
                pltpu.sync_copy(
                    acc.at[pl.ds(TAIL0, TAILN)],
                    out_hbm.at[cc, pl.ds(TAIL0, TAILN)],
                )

            plsc.subcore_barrier()

    return sc_agg


ROW_BLK = 2000


def _k1_body(z_ref, w_ref, b_ref, out_ref, s1_ref, s2_ref, *, C):
    i = pl.program_id(0)
    acc = jnp.dot(z_ref[0], w_ref[0], preferred_element_type=jnp.float32)
    for c in range(1, C):
        acc += jnp.dot(z_ref[c], w_ref[c], preferred_element_type=jnp.float32)
    z = acc + b_ref[...]
    out_ref[...] = z
    p1 = jnp.sum(z.reshape(ROW_BLK // 8, 8, D_H), axis=0)
    p2 = jnp.sum((z * z).reshape(ROW_BLK // 8, 8, D_H), axis=0)

    @pl.when(i == 0)
    def _():
        s1_ref[...] = p1
        s2_ref[...] = p2

    @pl.when(i > 0)
    def _():
        s1_ref[...] += p1
        s2_ref[...] += p2


def _k1(z_chunks, w1r, b1):
    C = z_chunks.shape[0]
    grid = (N // ROW_BLK,)
    return pl.pallas_call(
        functools.partial(_k1_body, C=C),
        grid=grid,
        in_specs=[
            pl.BlockSpec((C, ROW_BLK, 128), lambda i: (0, i, 0)),
            pl.BlockSpec((C, 128, D_H), lambda i: (0, 0, 0)),
            pl.BlockSpec((1, D_H), lambda i: (0, 0)),
        ],
        out_specs=[
            pl.BlockSpec((ROW_BLK, D_H), lambda i: (i, 0)),
            pl.BlockSpec((8, D_H), lambda i: (0, 0)),
            pl.BlockSpec((8, D_H), lambda i: (0, 0)),
        ],
        out_shape=[
            jax.ShapeDtypeStruct((N, D_H), jnp.float32),
            jax.ShapeDtypeStruct((8, D_H), jnp.float32),
            jax.ShapeDtypeStruct((8, D_H), jnp.float32),
        ],
    )(z_chunks, w1r, b1)


def _k2_body(z_ref, s1_ref, s2_ref, g_ref, be_ref, w2_ref, b2_ref, batch_ref,
             h_ref, pool_ref):
    i = pl.program_id(0)
    s1 = jnp.sum(s1_ref[...], axis=0, keepdims=True)
    s2 = jnp.sum(s2_ref[...], axis=0, keepdims=True)
    mean = s1 * (1.0 / N)
    var = s2 * (1.0 / N) - mean * mean
    inv = lax.rsqrt(var + 1e-5)
    scale = g_ref[...] * inv
    shift = be_ref[...] - mean * scale
    z = z_ref[...]
    r = jnp.maximum(z * scale + shift, 0.0)
    h = jnp.dot(r, w2_ref[...], preferred_element_type=jnp.float32) + b2_ref[...]
    h = jnp.maximum(h, 0.0)
    for c in range(D_H // 128):
        h_ref[c] = h[:, c * 128:(c + 1) * 128]
    onehot = (batch_ref[...] == lax.broadcasted_iota(jnp.int32, (ROW_BLK, G), 1)
              ).astype(jnp.float32)
    pp = lax.dot_general(onehot, h, dimension_numbers=(((0,), (0,)), ((), ())),
                         preferred_element_type=jnp.float32)

    @pl.when(i == 0)
    def _():
        pool_ref[...] = pp

    @pl.when(i > 0)
    def _():
        pool_ref[...] += pp


def _k2(z, s1, s2, g, be, w2, b2, batch2):
    grid = (N // ROW_BLK,)
    return pl.pallas_call(
        _k2_body,
        grid=grid,
        in_specs=[
            pl.BlockSpec((ROW_BLK, D_H), lambda i: (i, 0)),
            pl.BlockSpec((8, D_H), lambda i: (0, 0)),
            pl.BlockSpec((8, D_H), lambda i: (0, 0)),
            pl.BlockSpec((1, D_H), lambda i: (0, 0)),
            pl.BlockSpec((1, D_H), lambda i: (0, 0)),
            pl.BlockSpec((D_H, D_H), lambda i: (0, 0)),
            pl.BlockSpec((1, D_H), lambda i: (0, 0)),
            pl.BlockSpec((ROW_BLK, 1), lambda i: (i, 0)),
        ],
        out_specs=[
            pl.BlockSpec((D_H // 128, ROW_BLK, 128), lambda i: (0, i, 0)),
            pl.BlockSpec((G, D_H), lambda i: (0, 0)),
        ],
        out_shape=[
            jax.ShapeDtypeStruct((D_H // 128, N, 128), jnp.float32),
            jax.ShapeDtypeStruct((G, D_H), jnp.float32),
        ],
    )(z, s1, s2, g, be, w2, b2, batch2)


def _k3_body(p_ref, w_ref, b_ref, out_ref):
    o = jnp.sum(p_ref[...], axis=0)
    logits = jnp.dot(o, w_ref[...], preferred_element_type=jnp.float32) + b_ref[...]
    m = jnp.max(logits, axis=1, keepdims=True)
    e = jnp.exp(logits - m)
    s = jnp.sum(e, axis=1, keepdims=True)
    out_ref[...] = logits - m - jnp.log(s)


def _k3(pools, w_out, b_out):
    return pl.pallas_call(
        _k3_body,
        out_shape=jax.ShapeDtypeStruct((G, D_OUT), jnp.float32),
    )(pools, w_out, b_out)


def kernel(x, params, edge_index, batch):
    src = edge_index[0]
    dst = edge_index[1]
    batch2 = batch.reshape(N, 1)
    hc = x.reshape(N, 2, 128).transpose(1, 0, 2).reshape(2 * N, 128)
    pools = []
    for l in range(L):
        p = params['layers'][l]
        C = hc.shape[0] // N
        srcc = (src[None, :]
                + N * jnp.arange(C, dtype=jnp.int32)[:, None]).reshape(C * E)
        z_chunks = _make_sc_agg(C)(hc, srcc, dst)
        w1r = p['W1'].reshape(C, 128, D_H)
        b1 = p['b1'].reshape(1, D_H)
        zz, s1, s2 = _k1(z_chunks, w1r, b1)
        hcs, pool = _k2(zz, s1, s2, p['g1'].reshape(1, D_H),
                        p['be1'].reshape(1, D_H), p['W2'],
                        p['b2'].reshape(1, D_H), batch2)
        hc = hcs.reshape((D_H // 128) * N, 128)
        pools.append(pool)
    pstack = jnp.stack(pools)
    return _k3(pstack, params['W_out'], params['b_out'].reshape(1, D_OUT))


# speedup vs baseline: 1.0942x; 1.0456x over previous
import functools

import jax
import jax.numpy as jnp
from jax import lax
from jax.experimental import pallas as pl
from jax.experimental.pallas import tpu as pltpu
from jax.experimental.pallas import tpu_sc as plsc

N = 10000
E = 160000
D_H = 512
D_OUT = 128
G = 64
L = 4

NUM_TILES = 16
EDGE_BATCH = 80


@functools.lru_cache(maxsize=None)
def _make_sc_agg(C):
    CPC = C // 2
    EPT = E // NUM_TILES
    NIT = EPT // EDGE_BATCH
    RPT = 624
    TAIL0 = RPT * NUM_TILES
    TAILN = N - TAIL0

    mesh = plsc.VectorSubcoreMesh(core_axis_name="c", subcore_axis_name="s")

    NSET = 5
    NB = 2 * NSET
    NBODY = NIT // NB
    NTAIL = NIT - NBODY * NB
    assert NTAIL in (0, NSET)
    NROW = 4

    @functools.partial(
        pl.kernel,
        out_type=jax.ShapeDtypeStruct((C, N, 128), jnp.float32),
        mesh=mesh,
        scratch_types=(
            [pltpu.VMEM((EDGE_BATCH,), jnp.int32)] * (4 * NSET)
            + [pltpu.VMEM((EDGE_BATCH, 128), jnp.float32)] * NROW
            + [pltpu.VMEM_SHARED((N, 128), jnp.float32)]
            + [pltpu.SemaphoreType.DMA] * (2 + 2 * NROW)
        ),
    )
    def sc_agg(hc_hbm, srcc_hbm, dst_hbm, out_hbm, *scratch):
        sidx = list(scratch[0:NSET]) + list(scratch[2 * NSET:3 * NSET])
        didx = list(scratch[NSET:2 * NSET]) + list(scratch[3 * NSET:4 * NSET])
        rows = scratch[4 * NSET:4 * NSET + NROW]
        acc = scratch[4 * NSET + NROW]
        isem0, isem1 = scratch[4 * NSET + NROW + 1:4 * NSET + NROW + 3]
        ssems = scratch[4 * NSET + NROW + 3:4 * NSET + NROW + 3 + NROW]
        gsems = scratch[4 * NSET + NROW + 3 + NROW:]
        core = lax.axis_index("c")
        sub = lax.axis_index("s")
        r0 = sub * RPT
        e0 = sub * EPT

        for j in range(CPC):
            cc = core * CPC + j

            def load_idx(batch0, half, isem):
                for b in range(NSET):
                    e = e0 + (batch0 + b) * EDGE_BATCH
                    pltpu.async_copy(
                        srcc_hbm.at[pl.ds(cc * E + e, EDGE_BATCH)],
                        sidx[half * NSET + b], isem)
                    pltpu.async_copy(
                        dst_hbm.at[pl.ds(e, EDGE_BATCH)],
                        didx[half * NSET + b], isem)

            def drain_idx(isem):
                for _ in range(2 * NSET):
                    pltpu.make_async_copy(
                        dst_hbm.at[pl.ds(0, EDGE_BATCH)], didx[0], isem
                    ).wait()

            pltpu.sync_copy(
                hc_hbm.at[pl.ds(cc * N + r0, RPT)],
                acc.at[pl.ds(r0, RPT)],
            )

            @pl.when(sub == NUM_TILES - 1)
            def _():
                pltpu.sync_copy(
                    hc_hbm.at[pl.ds(cc * N + TAIL0, TAILN)],
                    acc.at[pl.ds(TAIL0, TAILN)],
                )

            plsc.subcore_barrier()
            load_idx(0, 0, isem0)

            def run_batches(nb, half1_load_base):
                drain_idx(isem0)
                gd, sd = {}, {}
                for i in range(min(NROW, nb)):
                    gd[i] = pltpu.async_copy(
                        hc_hbm.at[sidx[i]], rows[i], gsems[i])
                if nb > NSET:
                    load_idx(half1_load_base, 1, isem1)
                for i in range(nb):
                    gd[i].wait()
                    sd[i] = pltpu.async_copy(
                        rows[i % NROW], acc.at[didx[i]], ssems[i % NROW],
                        add=True)
                    jj = i + NROW
                    if jj < nb:
                        if jj == NSET:
                            drain_idx(isem1)
                        sd[i].wait()
                        gd[jj] = pltpu.async_copy(
                            hc_hbm.at[sidx[jj]], rows[jj % NROW], gsems[jj % NROW])
                for i in range(max(0, nb - NROW), nb):
                    sd[i].wait()

            def body(t, _):
                base = t * NB
                run_batches(NB, base + NSET)

                @pl.when(t < NBODY - 1)
                def _():
                    load_idx(base + NB, 0, isem0)

                return ()

            lax.fori_loop(0, NBODY, body, ())
            if NTAIL:
                load_idx(NBODY * NB, 0, isem0)
                run_batches(NTAIL, 0)
            plsc.subcore_barrier()
            pltpu.sync_copy(
                acc.at[pl.ds(r0, RPT)], out_hbm.at[cc, pl.ds(r0, RPT)]
            )

            @pl.when(sub == NUM_TILES - 1)
            def _():
                pltpu.sync_copy(
                    acc.at[pl.ds(TAIL0, TAILN)],
                    out_hbm.at[cc, pl.ds(TAIL0, TAILN)],
                )

            plsc.subcore_barrier()

    return sc_agg


ROW_BLK = 2000


def _k1_body(z_ref, w_ref, b_ref, out_ref, s1_ref, s2_ref, *, C):
    i = pl.program_id(0)
    acc = jnp.dot(z_ref[0], w_ref[0], preferred_element_type=jnp.float32)
    for c in range(1, C):
        acc += jnp.dot(z_ref[c], w_ref[c], preferred_element_type=jnp.float32)
    z = acc + b_ref[...]
    out_ref[...] = z
    p1 = jnp.sum(z.reshape(ROW_BLK // 8, 8, D_H), axis=0)
    p2 = jnp.sum((z * z).reshape(ROW_BLK // 8, 8, D_H), axis=0)

    @pl.when(i == 0)
    def _():
        s1_ref[...] = p1
        s2_ref[...] = p2

    @pl.when(i > 0)
    def _():
        s1_ref[...] += p1
        s2_ref[...] += p2


def _k1(z_chunks, w1r, b1):
    C = z_chunks.shape[0]
    grid = (N // ROW_BLK,)
    return pl.pallas_call(
        functools.partial(_k1_body, C=C),
        grid=grid,
        in_specs=[
            pl.BlockSpec((C, ROW_BLK, 128), lambda i: (0, i, 0)),
            pl.BlockSpec((C, 128, D_H), lambda i: (0, 0, 0)),
            pl.BlockSpec((1, D_H), lambda i: (0, 0)),
        ],
        out_specs=[
            pl.BlockSpec((ROW_BLK, D_H), lambda i: (i, 0)),
            pl.BlockSpec((8, D_H), lambda i: (0, 0)),
            pl.BlockSpec((8, D_H), lambda i: (0, 0)),
        ],
        out_shape=[
            jax.ShapeDtypeStruct((N, D_H), jnp.float32),
            jax.ShapeDtypeStruct((8, D_H), jnp.float32),
            jax.ShapeDtypeStruct((8, D_H), jnp.float32),
        ],
    )(z_chunks, w1r, b1)


def _k2_body(z_ref, s1_ref, s2_ref, g_ref, be_ref, w2_ref, b2_ref, batch_ref,
             h_ref, pool_ref):
    i = pl.program_id(0)
    s1 = jnp.sum(s1_ref[...], axis=0, keepdims=True)
    s2 = jnp.sum(s2_ref[...], axis=0, keepdims=True)
    mean = s1 * (1.0 / N)
    var = s2 * (1.0 / N) - mean * mean
    inv = lax.rsqrt(var + 1e-5)
    scale = g_ref[...] * inv
    shift = be_ref[...] - mean * scale
    z = z_ref[...]
    r = jnp.maximum(z * scale + shift, 0.0)
    h = jnp.dot(r, w2_ref[...], preferred_element_type=jnp.float32) + b2_ref[...]
    h = jnp.maximum(h, 0.0)
    for c in range(D_H // 128):
        h_ref[c] = h[:, c * 128:(c + 1) * 128]
    onehot = (batch_ref[...] == lax.broadcasted_iota(jnp.int32, (ROW_BLK, G), 1)
              ).astype(jnp.float32)
    pp = lax.dot_general(onehot, h, dimension_numbers=(((0,), (0,)), ((), ())),
                         preferred_element_type=jnp.float32)

    @pl.when(i == 0)
    def _():
        pool_ref[...] = pp

    @pl.when(i > 0)
    def _():
        pool_ref[...] += pp


def _k2(z, s1, s2, g, be, w2, b2, batch2):
    grid = (N // ROW_BLK,)
    return pl.pallas_call(
        _k2_body,
        grid=grid,
        in_specs=[
            pl.BlockSpec((ROW_BLK, D_H), lambda i: (i, 0)),
            pl.BlockSpec((8, D_H), lambda i: (0, 0)),
            pl.BlockSpec((8, D_H), lambda i: (0, 0)),
            pl.BlockSpec((1, D_H), lambda i: (0, 0)),
            pl.BlockSpec((1, D_H), lambda i: (0, 0)),
            pl.BlockSpec((D_H, D_H), lambda i: (0, 0)),
            pl.BlockSpec((1, D_H), lambda i: (0, 0)),
            pl.BlockSpec((ROW_BLK, 1), lambda i: (i, 0)),
        ],
        out_specs=[
            pl.BlockSpec((D_H // 128, ROW_BLK, 128), lambda i: (0, i, 0)),
            pl.BlockSpec((G, D_H), lambda i: (0, 0)),
        ],
        out_shape=[
            jax.ShapeDtypeStruct((D_H // 128, N, 128), jnp.float32),
            jax.ShapeDtypeStruct((G, D_H), jnp.float32),
        ],
    )(z, s1, s2, g, be, w2, b2, batch2)


def _k3_body(p_ref, w_ref, b_ref, out_ref):
    o = jnp.sum(p_ref[...], axis=0)
    logits = jnp.dot(o, w_ref[...], preferred_element_type=jnp.float32) + b_ref[...]
    m = jnp.max(logits, axis=1, keepdims=True)
    e = jnp.exp(logits - m)
    s = jnp.sum(e, axis=1, keepdims=True)
    out_ref[...] = logits - m - jnp.log(s)


def _k3(pools, w_out, b_out):
    return pl.pallas_call(
        _k3_body,
        out_shape=jax.ShapeDtypeStruct((G, D_OUT), jnp.float32),
    )(pools, w_out, b_out)


def kernel(x, params, edge_index, batch):
    src = edge_index[0]
    dst = edge_index[1]
    batch2 = batch.reshape(N, 1)
    hc = x.reshape(N, 2, 128).transpose(1, 0, 2).reshape(2 * N, 128)
    pools = []
    for l in range(L):
        p = params['layers'][l]
        C = hc.shape[0] // N
        srcc = (src[None, :]
                + N * jnp.arange(C, dtype=jnp.int32)[:, None]).reshape(C * E)
        z_chunks = _make_sc_agg(C)(hc, srcc, dst)
        w1r = p['W1'].reshape(C, 128, D_H)
        b1 = p['b1'].reshape(1, D_H)
        zz, s1, s2 = _k1(z_chunks, w1r, b1)
        hcs, pool = _k2(zz, s1, s2, p['g1'].reshape(1, D_H),
                        p['be1'].reshape(1, D_H), p['W2'],
                        p['b2'].reshape(1, D_H), batch2)
        hc = hcs.reshape((D_H // 128) * N, 128)
        pools.append(pool)
    pstack = jnp.stack(pools)
    return _k3(pstack, params['W_out'], params['b_out'].reshape(1, D_OUT))


# speedup vs baseline: 1.0954x; 1.0011x over previous
import functools

import jax
import jax.numpy as jnp
from jax import lax
from jax.experimental import pallas as pl
from jax.experimental.pallas import tpu as pltpu
from jax.experimental.pallas import tpu_sc as plsc

N = 10000
E = 160000
D_H = 512
D_OUT = 128
G = 64
L = 4

NUM_TILES = 16
EDGE_BATCH = 80


@functools.lru_cache(maxsize=None)
def _make_sc_agg(C):
    CPC = C // 2
    EPT = E // NUM_TILES
    NIT = EPT // EDGE_BATCH
    RPT = 624
    TAIL0 = RPT * NUM_TILES
    TAILN = N - TAIL0

    mesh = plsc.VectorSubcoreMesh(core_axis_name="c", subcore_axis_name="s")

    NSET = 5
    NB = 2 * NSET
    NBODY = NIT // NB
    NTAIL = NIT - NBODY * NB
    assert NTAIL in (0, NSET)
    NROW = 4

    @functools.partial(
        pl.kernel,
        out_type=jax.ShapeDtypeStruct((C, N, 128), jnp.float32),
        mesh=mesh,
        scratch_types=(
            [pltpu.VMEM((EDGE_BATCH,), jnp.int32)] * (4 * NSET)
            + [pltpu.VMEM((EDGE_BATCH, 128), jnp.float32)] * NROW
            + [pltpu.VMEM_SHARED((N, 128), jnp.float32)]
            + [pltpu.SemaphoreType.DMA] * (2 + 2 * NROW)
        ),
    )
    def sc_agg(hc_hbm, srcc_hbm, dst_hbm, out_hbm, *scratch):
        sidx = list(scratch[0:NSET]) + list(scratch[2 * NSET:3 * NSET])
        didx = list(scratch[NSET:2 * NSET]) + list(scratch[3 * NSET:4 * NSET])
        rows = scratch[4 * NSET:4 * NSET + NROW]
        acc = scratch[4 * NSET + NROW]
        isem0, isem1 = scratch[4 * NSET + NROW + 1:4 * NSET + NROW + 3]
        ssems = scratch[4 * NSET + NROW + 3:4 * NSET + NROW + 3 + NROW]
        gsems = scratch[4 * NSET + NROW + 3 + NROW:]
        core = lax.axis_index("c")
        sub = lax.axis_index("s")
        r0 = sub * RPT
        e0 = sub * EPT

        for j in range(CPC):
            cc = core * CPC + j

            def load_idx(batch0, half, isem):
                for b in range(NSET):
                    e = e0 + (batch0 + b) * EDGE_BATCH
                    pltpu.async_copy(
                        srcc_hbm.at[pl.ds(cc * E + e, EDGE_BATCH)],
                        sidx[half * NSET + b], isem)
                    pltpu.async_copy(
                        dst_hbm.at[pl.ds(e, EDGE_BATCH)],
                        didx[half * NSET + b], isem)

            def drain_idx(isem):
                for _ in range(2 * NSET):
                    pltpu.make_async_copy(
                        dst_hbm.at[pl.ds(0, EDGE_BATCH)], didx[0], isem
                    ).wait()

            pltpu.sync_copy(
                hc_hbm.at[pl.ds(cc * N + r0, RPT)],
                acc.at[pl.ds(r0, RPT)],
            )

            @pl.when(sub == NUM_TILES - 1)
            def _():
                pltpu.sync_copy(
                    hc_hbm.at[pl.ds(cc * N + TAIL0, TAILN)],
                    acc.at[pl.ds(TAIL0, TAILN)],
                )

            plsc.subcore_barrier()
            load_idx(0, 0, isem0)

            def run_batches(nb, half1_load_base):
                drain_idx(isem0)
                gd, sd = {}, {}
                for i in range(min(NROW, nb)):
                    gd[i] = pltpu.async_copy(
                        hc_hbm.at[sidx[i]], rows[i], gsems[i])
                if nb > NSET:
                    load_idx(half1_load_base, 1, isem1)
                for i in range(nb):
                    gd[i].wait()
                    sd[i] = pltpu.async_copy(
                        rows[i % NROW], acc.at[didx[i]], ssems[i % NROW],
                        add=True)
                    jj = i + NROW
                    if jj < nb:
                        if jj == NSET:
                            drain_idx(isem1)
                        sd[i].wait()
                        gd[jj] = pltpu.async_copy(
                            hc_hbm.at[sidx[jj]], rows[jj % NROW], gsems[jj % NROW])
                for i in range(max(0, nb - NROW), nb):
                    sd[i].wait()

            def body(t, _):
                base = t * NB
                run_batches(NB, base + NSET)

                @pl.when(t < NBODY - 1)
                def _():
                    load_idx(base + NB, 0, isem0)

                return ()

            lax.fori_loop(0, NBODY, body, ())
            if NTAIL:
                load_idx(NBODY * NB, 0, isem0)
                run_batches(NTAIL, 0)
            plsc.subcore_barrier()
            pltpu.sync_copy(
                acc.at[pl.ds(r0, RPT)], out_hbm.at[cc, pl.ds(r0, RPT)]
            )

            @pl.when(sub == NUM_TILES - 1)
            def _():
                pltpu.sync_copy(
                    acc.at[pl.ds(TAIL0, TAILN)],
                    out_hbm.at[cc, pl.ds(TAIL0, TAILN)],
                )

            plsc.subcore_barrier()

    return sc_agg


ROW_BLK = 5000


def _k1_body(z_ref, w_ref, b_ref, out_ref, s1_ref, s2_ref, *, C):
    i = pl.program_id(0)
    acc = jnp.dot(z_ref[0], w_ref[0], preferred_element_type=jnp.float32)
    for c in range(1, C):
        acc += jnp.dot(z_ref[c], w_ref[c], preferred_element_type=jnp.float32)
    z = acc + b_ref[...]
    out_ref[...] = z
    p1 = jnp.sum(z.reshape(ROW_BLK // 8, 8, D_H), axis=0)
    p2 = jnp.sum((z * z).reshape(ROW_BLK // 8, 8, D_H), axis=0)

    @pl.when(i == 0)
    def _():
        s1_ref[...] = p1
        s2_ref[...] = p2

    @pl.when(i > 0)
    def _():
        s1_ref[...] += p1
        s2_ref[...] += p2


def _k1(z_chunks, w1r, b1):
    C = z_chunks.shape[0]
    grid = (N // ROW_BLK,)
    return pl.pallas_call(
        functools.partial(_k1_body, C=C),
        grid=grid,
        in_specs=[
            pl.BlockSpec((C, ROW_BLK, 128), lambda i: (0, i, 0)),
            pl.BlockSpec((C, 128, D_H), lambda i: (0, 0, 0)),
            pl.BlockSpec((1, D_H), lambda i: (0, 0)),
        ],
        out_specs=[
            pl.BlockSpec((ROW_BLK, D_H), lambda i: (i, 0)),
            pl.BlockSpec((8, D_H), lambda i: (0, 0)),
            pl.BlockSpec((8, D_H), lambda i: (0, 0)),
        ],
        out_shape=[
            jax.ShapeDtypeStruct((N, D_H), jnp.float32),
            jax.ShapeDtypeStruct((8, D_H), jnp.float32),
            jax.ShapeDtypeStruct((8, D_H), jnp.float32),
        ],
    )(z_chunks, w1r, b1)


def _k2_body(z_ref, s1_ref, s2_ref, g_ref, be_ref, w2_ref, b2_ref, batch_ref,
             h_ref, pool_ref):
    i = pl.program_id(0)
    s1 = jnp.sum(s1_ref[...], axis=0, keepdims=True)
    s2 = jnp.sum(s2_ref[...], axis=0, keepdims=True)
    mean = s1 * (1.0 / N)
    var = s2 * (1.0 / N) - mean * mean
    inv = lax.rsqrt(var + 1e-5)
    scale = g_ref[...] * inv
    shift = be_ref[...] - mean * scale
    z = z_ref[...]
    r = jnp.maximum(z * scale + shift, 0.0)
    h = jnp.dot(r, w2_ref[...], preferred_element_type=jnp.float32) + b2_ref[...]
    h = jnp.maximum(h, 0.0)
    for c in range(D_H // 128):
        h_ref[c] = h[:, c * 128:(c + 1) * 128]
    onehot = (batch_ref[...] == lax.broadcasted_iota(jnp.int32, (ROW_BLK, G), 1)
              ).astype(jnp.float32)
    pp = lax.dot_general(onehot, h, dimension_numbers=(((0,), (0,)), ((), ())),
                         preferred_element_type=jnp.float32)

    @pl.when(i == 0)
    def _():
        pool_ref[...] = pp

    @pl.when(i > 0)
    def _():
        pool_ref[...] += pp


def _k2(z, s1, s2, g, be, w2, b2, batch2):
    grid = (N // ROW_BLK,)
    return pl.pallas_call(
        _k2_body,
        grid=grid,
        in_specs=[
            pl.BlockSpec((ROW_BLK, D_H), lambda i: (i, 0)),
            pl.BlockSpec((8, D_H), lambda i: (0, 0)),
            pl.BlockSpec((8, D_H), lambda i: (0, 0)),
            pl.BlockSpec((1, D_H), lambda i: (0, 0)),
            pl.BlockSpec((1, D_H), lambda i: (0, 0)),
            pl.BlockSpec((D_H, D_H), lambda i: (0, 0)),
            pl.BlockSpec((1, D_H), lambda i: (0, 0)),
            pl.BlockSpec((ROW_BLK, 1), lambda i: (i, 0)),
        ],
        out_specs=[
            pl.BlockSpec((D_H // 128, ROW_BLK, 128), lambda i: (0, i, 0)),
            pl.BlockSpec((G, D_H), lambda i: (0, 0)),
        ],
        out_shape=[
            jax.ShapeDtypeStruct((D_H // 128, N, 128), jnp.float32),
            jax.ShapeDtypeStruct((G, D_H), jnp.float32),
        ],
    )(z, s1, s2, g, be, w2, b2, batch2)


def _k3_body(p_ref, w_ref, b_ref, out_ref):
    o = jnp.sum(p_ref[...], axis=0)
    logits = jnp.dot(o, w_ref[...], preferred_element_type=jnp.float32) + b_ref[...]
    m = jnp.max(logits, axis=1, keepdims=True)
    e = jnp.exp(logits - m)
    s = jnp.sum(e, axis=1, keepdims=True)
    out_ref[...] = logits - m - jnp.log(s)


def _k3(pools, w_out, b_out):
    return pl.pallas_call(
        _k3_body,
        out_shape=jax.ShapeDtypeStruct((G, D_OUT), jnp.float32),
    )(pools, w_out, b_out)


def kernel(x, params, edge_index, batch):
    src = edge_index[0]
    dst = edge_index[1]
    batch2 = batch.reshape(N, 1)
    hc = x.reshape(N, 2, 128).transpose(1, 0, 2).reshape(2 * N, 128)
    pools = []
    for l in range(L):
        p = params['layers'][l]
        C = hc.shape[0] // N
        srcc = (src[None, :]
                + N * jnp.arange(C, dtype=jnp.int32)[:, None]).reshape(C * E)
        z_chunks = _make_sc_agg(C)(hc, srcc, dst)
        w1r = p['W1'].reshape(C, 128, D_H)
        b1 = p['b1'].reshape(1, D_H)
        zz, s1, s2 = _k1(z_chunks, w1r, b1)
        hcs, pool = _k2(zz, s1, s2, p['g1'].reshape(1, D_H),
                        p['be1'].reshape(1, D_H), p['W2'],
                        p['b2'].reshape(1, D_H), batch2)
        hc = hcs.reshape((D_H // 128) * N, 128)
        pools.append(pool)
    pstack = jnp.stack(pools)
    return _k3(pstack, params['W_out'], params['b_out'].reshape(1, D_OUT))
